# Initial kernel scaffold; baseline (speedup 1.0000x reference)
import jax
import jax.numpy as jnp
from jax.experimental import pallas as pl


def kernel(features, confidence, mem_features, mem_confidences):
    raise NotImplementedError("write your pallas kernel here")


# speedup vs baseline: 2.4167x; 2.4167x over previous
import functools

import jax
import jax.numpy as jnp
from jax import lax
from jax.experimental import pallas as pl
from jax.experimental.pallas import tpu as pltpu
from jax.experimental.pallas import tpu_sc as plsc

BATCH = 16384
CHANNELS = 2048
MEM = 16384
CONF_T = 0.5

NC, NS, LANES = 2, 16, 16
NW = NC * NS
RPW = BATCH // NW
VPW = RPW // LANES
GCH = 16


def _sc_body(feat_hbm, conf_hbm, out_hbm,
             conf_v, idx_v, buf_a, buf_b, zbuf,
             sem_ga, sem_gb, sem_wa, sem_wb, sem_z):
    wid = lax.axis_index("c") * NS + lax.axis_index("s")

    pltpu.sync_copy(conf_hbm, conf_v)

    zerosf = jnp.zeros((LANES,), jnp.float32)

    def zrow(j, _):
        def zcol(k, _):
            zbuf[j, pl.ds(k * LANES, LANES)] = zerosf
            return 0
        return lax.fori_loop(0, CHANNELS // LANES, zcol, 0)
    lax.fori_loop(0, GCH, zrow, 0)

    zeros16 = jnp.zeros((LANES,), jnp.int32)

    def cnt_body(i, acc):
        v = conf_v[pl.ds(i * LANES, LANES)]
        return acc + jnp.where(v > CONF_T, 1, 0).astype(jnp.int32)

    acc_pre = lax.fori_loop(0, wid * VPW, cnt_body, zeros16)
    acc_own = lax.fori_loop(wid * VPW, (wid + 1) * VPW, cnt_body, zeros16)
    acc_post = lax.fori_loop((wid + 1) * VPW, BATCH // LANES, cnt_body, zeros16)
    astart = jnp.sum(acc_pre)
    nacc = jnp.sum(acc_own)
    k_total = astart + nacc + jnp.sum(acc_post)
    zstart = k_total + wid * RPW - astart

    def zidx(i, _):
        idx_v[pl.ds(i * LANES, LANES)] = zeros16
        return 0
    lax.fori_loop(0, VPW, zidx, 0)

    lane = lax.iota(jnp.int32, 16)
    own0 = wid * RPW
    cnt = zeros16
    for ci in range(VPW):
        v = conf_v[pl.ds(own0 + ci * LANES, LANES)]
        m = v > CONF_T
        within = plsc.cumsum(jnp.where(m, 1, 0).astype(jnp.int32)) - 1
        plsc.store_scatter(idx_v, [cnt + within], lane + (own0 + ci * LANES),
                           mask=m)
        cnt = cnt + plsc.all_reduce_population_count(m)

    nrej = RPW - nacc
    z_full = nrej // GCH
    zrem = nrej - z_full * GCH

    def zbody(i, _):
        pltpu.async_copy(zbuf, out_hbm.at[pl.ds(zstart + i * GCH, GCH)],
                         sem_z)
        return 0
    lax.fori_loop(0, z_full, zbody, 0)

    @pl.when(zrem > 0)
    def _():
        def zrbody(r, _):
            pltpu.async_copy(zbuf.at[pl.ds(0, 1)],
                             out_hbm.at[pl.ds(zstart + z_full * GCH + r, 1)],
                             sem_z)
            return 0
        lax.fori_loop(0, zrem, zrbody, 0)

    n_full = nacc // GCH
    rem = nacc - n_full * GCH

    def _wait_gather(x_buf, x_sem):
        pltpu.make_async_copy(feat_hbm.at[pl.ds(0, GCH)], x_buf, x_sem).wait()

    def _wait_write(x_buf, x_sem):
        pltpu.make_async_copy(x_buf, out_hbm.at[pl.ds(0, GCH)], x_sem).wait()

    def _step(i, x_buf, x_gsem, x_wsem, y_buf, y_gsem, y_wsem):
        _wait_gather(x_buf, x_gsem)
        pltpu.async_copy(x_buf, out_hbm.at[pl.ds(astart + i * GCH, GCH)],
                         x_wsem)

        @pl.when(i + 1 < n_full)
        def _():
            @pl.when(i >= 1)
            def _():
                _wait_write(y_buf, y_wsem)
            ivec = idx_v[pl.ds((i + 1) * GCH, GCH)]
            pltpu.async_copy(feat_hbm.at[ivec], y_buf, y_gsem)

    @pl.when(n_full > 0)
    def _():
        ivec0 = idx_v[pl.ds(0, GCH)]
        pltpu.async_copy(feat_hbm.at[ivec0], buf_a, sem_ga)

    def gbody(i, _):
        @pl.when(i % 2 == 0)
        def _():
            _step(i, buf_a, sem_ga, sem_wa, buf_b, sem_gb, sem_wb)

        @pl.when(i % 2 == 1)
        def _():
            _step(i, buf_b, sem_gb, sem_wb, buf_a, sem_ga, sem_wa)
        return 0
    lax.fori_loop(0, n_full, gbody, 0)

    @pl.when(n_full >= 1)
    def _():
        @pl.when((n_full - 1) % 2 == 0)
        def _():
            _wait_write(buf_a, sem_wa)

        @pl.when((n_full - 1) % 2 == 1)
        def _():
            _wait_write(buf_b, sem_wb)

    @pl.when(n_full >= 2)
    def _():
        @pl.when((n_full - 2) % 2 == 0)
        def _():
            _wait_write(buf_a, sem_wa)

        @pl.when((n_full - 2) % 2 == 1)
        def _():
            _wait_write(buf_b, sem_wb)

    @pl.when(rem > 0)
    def _():
        ivec = idx_v[pl.ds(n_full * GCH, GCH)]
        pltpu.async_copy(feat_hbm.at[ivec], buf_a, sem_ga).wait()

        def wbody(r, _):
            pltpu.async_copy(buf_a.at[pl.ds(r, 1)],
                             out_hbm.at[pl.ds(astart + n_full * GCH + r, 1)],
                             sem_wa)
            return 0
        lax.fori_loop(0, rem, wbody, 0)

        def wdrain(r, _):
            pltpu.make_async_copy(buf_a.at[pl.ds(0, 1)],
                                  out_hbm.at[pl.ds(0, 1)], sem_wa).wait()
            return 0
        lax.fori_loop(0, rem, wdrain, 0)

    def zdrain(i, _):
        pltpu.make_async_copy(zbuf, out_hbm.at[pl.ds(0, GCH)], sem_z).wait()
        return 0
    lax.fori_loop(0, z_full, zdrain, 0)

    @pl.when(zrem > 0)
    def _():
        def zrdrain(r, _):
            pltpu.make_async_copy(zbuf.at[pl.ds(0, 1)],
                                  out_hbm.at[pl.ds(0, 1)], sem_z).wait()
            return 0
        lax.fori_loop(0, zrem, zrdrain, 0)


def _sc_compact(features, confidence):
    mesh = plsc.VectorSubcoreMesh(core_axis_name="c", subcore_axis_name="s",
                                  num_cores=NC, num_subcores=NS)
    return pl.kernel(
        _sc_body,
        out_type=jax.ShapeDtypeStruct((MEM + 2, CHANNELS), jnp.float32),
        mesh=mesh,
        compiler_params=pltpu.CompilerParams(use_tc_tiling_on_sc=False, needs_layout_passes=False),
        scratch_types=[
            pltpu.VMEM((BATCH,), jnp.float32),
            pltpu.VMEM((RPW,), jnp.int32),
            pltpu.VMEM((GCH, CHANNELS), jnp.float32),
            pltpu.VMEM((GCH, CHANNELS), jnp.float32),
            pltpu.VMEM((GCH, CHANNELS), jnp.float32),
            pltpu.SemaphoreType.DMA,
            pltpu.SemaphoreType.DMA,
            pltpu.SemaphoreType.DMA,
            pltpu.SemaphoreType.DMA,
            pltpu.SemaphoreType.DMA,
        ],
    )(features, confidence)


STAT_BLK = 256
STAT_STEPS = BATCH // STAT_BLK


def _stats_body(conf_ref, feat_ref, out_ref, s1_ref, s2_ref, c_ref):
    i = pl.program_id(0)

    @pl.when(i == 0)
    def _():
        s1_ref[...] = jnp.zeros_like(s1_ref)
        s2_ref[...] = jnp.zeros_like(s2_ref)
        c_ref[0] = 0.0

    cw = jnp.squeeze(conf_ref[...], axis=0)
    w = jnp.where(cw > CONF_T, cw, 0.0)
    f = feat_ref[...]
    dot = functools.partial(lax.dot_general,
                            dimension_numbers=(((1,), (0,)), ((), ())),
                            preferred_element_type=jnp.float32)
    s1_ref[...] += dot(w, f)
    s2_ref[...] += dot(w, f * f)
    c_ref[0] += jnp.sum(w)

    @pl.when(i == STAT_STEPS - 1)
    def _():
        c = c_ref[0]
        t = c + 1e-8
        s1 = s1_ref[...]
        s2 = s2_ref[...]
        mean = s1 / t
        var = (s2 - 2.0 * mean * s1 + mean * mean * c) / t
        out_ref[...] = jnp.concatenate(
            [mean, jnp.sqrt(var + 1e-8)], axis=0)


def _tc_stats(features, confidence):
    conf3 = confidence.reshape(STAT_STEPS, 1, STAT_BLK)
    return pl.pallas_call(
        _stats_body,
        grid=(STAT_STEPS,),
        in_specs=[
            pl.BlockSpec((1, 1, STAT_BLK), lambda i: (i, 0, 0)),
            pl.BlockSpec((STAT_BLK, CHANNELS), lambda i: (i, 0)),
        ],
        out_specs=pl.BlockSpec((2, CHANNELS), lambda i: (0, 0)),
        out_shape=jax.ShapeDtypeStruct((2, CHANNELS), jnp.float32),
        scratch_shapes=[
            pltpu.VMEM((1, CHANNELS), jnp.float32),
            pltpu.VMEM((1, CHANNELS), jnp.float32),
            pltpu.SMEM((1,), jnp.float32),
        ],
    )(conf3, features)


def _epi_body(table_ref, ms_ref, out_ref, sem):
    copy = pltpu.make_async_copy(ms_ref, out_ref.at[pl.ds(MEM, 2)], sem)
    copy.start()
    copy.wait()


def _tc_epilogue(table, meanstd):
    return pl.pallas_call(
        _epi_body,
        in_specs=[
            pl.BlockSpec(memory_space=pl.ANY),
            pl.BlockSpec(memory_space=pltpu.VMEM),
        ],
        out_specs=pl.BlockSpec(memory_space=pl.ANY),
        out_shape=jax.ShapeDtypeStruct((MEM + 2, CHANNELS), jnp.float32),
        scratch_shapes=[pltpu.SemaphoreType.DMA],
        input_output_aliases={0: 0},
    )(table, meanstd)


def kernel(features, confidence, mem_features, mem_confidences):
    table = _sc_compact(features, confidence)
    meanstd = _tc_stats(features, confidence)
    return _tc_epilogue(table, meanstd)


# speedup vs baseline: 6.4763x; 2.6798x over previous
import functools

import jax
import jax.numpy as jnp
from jax import lax
from jax.experimental import pallas as pl
from jax.experimental.pallas import tpu as pltpu
from jax.experimental.pallas import tpu_sc as plsc

BATCH = 16384
CHANNELS = 2048
MEM = 16384
CONF_T = 0.5

NC, NS, LANES = 2, 16, 16
NW = NC * NS
RPW = BATCH // NW
VPW = RPW // LANES
GCH = 16


def _sc_body(feat_hbm, conf_hbm, out_hbm,
             conf_v, idx_v, buf_a, buf_b, zbuf,
             sem_ga, sem_gb, sem_wa, sem_wb, sem_z):
    wid = lax.axis_index("s") * NC + lax.axis_index("c")
    own0 = wid * RPW

    pltpu.sync_copy(conf_hbm, conf_v)

    zerosf = jnp.zeros((LANES,), jnp.float32)

    def zrow(j, _):
        def zcol(k, _):
            zbuf[j, pl.ds(k * LANES, LANES)] = zerosf
            return 0
        return lax.fori_loop(0, CHANNELS // LANES, zcol, 0)
    lax.fori_loop(0, GCH, zrow, 0)

    zeros16 = jnp.zeros((LANES,), jnp.int32)

    def zidx(i, _):
        idx_v[pl.ds(i * LANES, LANES)] = zeros16
        return 0
    lax.fori_loop(0, VPW, zidx, 0)

    lane = lax.iota(jnp.int32, 16)

    def scan_body(i, cum):
        v = conf_v[pl.ds(i * LANES, LANES)]
        m = v > CONF_T
        within = plsc.cumsum(jnp.where(m, 1, 0).astype(jnp.int32))
        dest = cum + within - (1 + own0)
        mk = m & (dest >= 0) & (dest < RPW)
        plsc.store_scatter(idx_v, [dest], lane + i * LANES, mask=mk)
        return cum + plsc.all_reduce_population_count(m)
    cum_fin = lax.fori_loop(0, BATCH // LANES, scan_body, zeros16)
    k_total = jnp.max(cum_fin)
    ndata = jnp.clip(k_total - own0, 0, RPW)

    n_full = ndata // GCH
    rem = ndata - n_full * GCH
    zc0 = n_full + jnp.where(rem > 0, 1, 0)

    def zbody(i, _):
        pltpu.async_copy(zbuf, out_hbm.at[pl.ds(own0 + i * GCH, GCH)],
                         sem_z)
        return 0
    lax.fori_loop(zc0, RPW // GCH, zbody, 0)


    def _wait_gather(x_buf, x_sem):
        pltpu.make_async_copy(feat_hbm.at[pl.ds(0, GCH)], x_buf, x_sem).wait()

    def _wait_write(x_buf, x_sem):
        pltpu.make_async_copy(x_buf, out_hbm.at[pl.ds(0, GCH)], x_sem).wait()

    def _step(i, x_buf, x_gsem, x_wsem, y_buf, y_gsem, y_wsem):
        _wait_gather(x_buf, x_gsem)
        pltpu.async_copy(x_buf, out_hbm.at[pl.ds(own0 + i * GCH, GCH)],
                         x_wsem)

        @pl.when(i + 1 < n_full)
        def _():
            @pl.when(i >= 1)
            def _():
                _wait_write(y_buf, y_wsem)
            ivec = idx_v[pl.ds((i + 1) * GCH, GCH)]
            pltpu.async_copy(feat_hbm.at[ivec], y_buf, y_gsem)

    @pl.when(n_full > 0)
    def _():
        ivec0 = idx_v[pl.ds(0, GCH)]
        pltpu.async_copy(feat_hbm.at[ivec0], buf_a, sem_ga)

    def gbody(i, _):
        @pl.when(i % 2 == 0)
        def _():
            _step(i, buf_a, sem_ga, sem_wa, buf_b, sem_gb, sem_wb)

        @pl.when(i % 2 == 1)
        def _():
            _step(i, buf_b, sem_gb, sem_wb, buf_a, sem_ga, sem_wa)
        return 0
    lax.fori_loop(0, n_full, gbody, 0)

    @pl.when(n_full >= 1)
    def _():
        @pl.when((n_full - 1) % 2 == 0)
        def _():
            _wait_write(buf_a, sem_wa)

        @pl.when((n_full - 1) % 2 == 1)
        def _():
            _wait_write(buf_b, sem_wb)

    @pl.when(n_full >= 2)
    def _():
        @pl.when((n_full - 2) % 2 == 0)
        def _():
            _wait_write(buf_a, sem_wa)

        @pl.when((n_full - 2) % 2 == 1)
        def _():
            _wait_write(buf_b, sem_wb)

    @pl.when(rem > 0)
    def _():
        ivec = idx_v[pl.ds(n_full * GCH, GCH)]
        pltpu.async_copy(feat_hbm.at[ivec], buf_a, sem_ga).wait()

        def zrow_tail(r, _):
            def zcol_tail(k, _):
                buf_a[r, pl.ds(k * LANES, LANES)] = zerosf
                return 0
            return lax.fori_loop(0, CHANNELS // LANES, zcol_tail, 0)
        lax.fori_loop(rem, GCH, zrow_tail, 0)

        pltpu.async_copy(buf_a, out_hbm.at[pl.ds(own0 + n_full * GCH, GCH)],
                         sem_wa).wait()

    def zdrain(i, _):
        pltpu.make_async_copy(zbuf, out_hbm.at[pl.ds(0, GCH)], sem_z).wait()
        return 0
    lax.fori_loop(zc0, RPW // GCH, zdrain, 0)


def _sc_compact(features, confidence):
    mesh = plsc.VectorSubcoreMesh(core_axis_name="c", subcore_axis_name="s",
                                  num_cores=NC, num_subcores=NS)
    return pl.kernel(
        _sc_body,
        out_type=jax.ShapeDtypeStruct((MEM + 2, CHANNELS), jnp.float32),
        mesh=mesh,
        compiler_params=pltpu.CompilerParams(use_tc_tiling_on_sc=True,
                                             needs_layout_passes=False),
        scratch_types=[
            pltpu.VMEM((BATCH,), jnp.float32),
            pltpu.VMEM((RPW,), jnp.int32),
            pltpu.VMEM((GCH, CHANNELS), jnp.float32),
            pltpu.VMEM((GCH, CHANNELS), jnp.float32),
            pltpu.VMEM((GCH, CHANNELS), jnp.float32),
            pltpu.SemaphoreType.DMA,
            pltpu.SemaphoreType.DMA,
            pltpu.SemaphoreType.DMA,
            pltpu.SemaphoreType.DMA,
            pltpu.SemaphoreType.DMA,
        ],
    )(features, confidence)


STAT_BLK = 1024
STAT_STEPS = BATCH // STAT_BLK


def _stats_body(conf_ref, feat_ref, out_ref, s1_ref, s2_ref, c_ref):
    i = pl.program_id(0)

    @pl.when(i == 0)
    def _():
        s1_ref[...] = jnp.zeros_like(s1_ref)
        s2_ref[...] = jnp.zeros_like(s2_ref)
        c_ref[0] = 0.0

    cw = jnp.squeeze(conf_ref[...], axis=0)
    w = jnp.where(cw > CONF_T, cw, 0.0)
    f = feat_ref[...]
    dot = functools.partial(lax.dot_general,
                            dimension_numbers=(((1,), (0,)), ((), ())),
                            preferred_element_type=jnp.float32)
    s1_ref[...] += dot(w, f)
    s2_ref[...] += dot(w, f * f)
    c_ref[0] += jnp.sum(w)

    @pl.when(i == STAT_STEPS - 1)
    def _():
        c = c_ref[0]
        t = c + 1e-8
        s1 = s1_ref[...]
        s2 = s2_ref[...]
        mean = s1 / t
        var = (s2 - 2.0 * mean * s1 + mean * mean * c) / t
        out_ref[...] = jnp.concatenate(
            [mean, jnp.sqrt(var + 1e-8)], axis=0)


def _tc_stats(features, confidence):
    conf3 = confidence.reshape(STAT_STEPS, 1, STAT_BLK)
    return pl.pallas_call(
        _stats_body,
        grid=(STAT_STEPS,),
        in_specs=[
            pl.BlockSpec((1, 1, STAT_BLK), lambda i: (i, 0, 0)),
            pl.BlockSpec((STAT_BLK, CHANNELS), lambda i: (i, 0)),
        ],
        out_specs=pl.BlockSpec((2, CHANNELS), lambda i: (0, 0)),
        out_shape=jax.ShapeDtypeStruct((2, CHANNELS), jnp.float32),
        scratch_shapes=[
            pltpu.VMEM((1, CHANNELS), jnp.float32),
            pltpu.VMEM((1, CHANNELS), jnp.float32),
            pltpu.SMEM((1,), jnp.float32),
        ],
    )(conf3, features)


def _epi_body(table_ref, ms_ref, out_ref, sem):
    copy = pltpu.make_async_copy(ms_ref, out_ref.at[pl.ds(MEM, 2)], sem)
    copy.start()
    copy.wait()


def _tc_epilogue(table, meanstd):
    return pl.pallas_call(
        _epi_body,
        in_specs=[
            pl.BlockSpec(memory_space=pl.ANY),
            pl.BlockSpec(memory_space=pltpu.VMEM),
        ],
        out_specs=pl.BlockSpec(memory_space=pl.ANY),
        out_shape=jax.ShapeDtypeStruct((MEM + 2, CHANNELS), jnp.float32),
        scratch_shapes=[pltpu.SemaphoreType.DMA],
        input_output_aliases={0: 0},
    )(table, meanstd)


def kernel(features, confidence, mem_features, mem_confidences):
    table = _sc_compact(features, confidence)
    meanstd = _tc_stats(features, confidence)
    return _tc_epilogue(table, meanstd)
